# baseline (speedup 1.0000x reference)
import jax
import jax.numpy as jnp
from jax.experimental import pallas as pl
from jax.experimental.pallas import tpu as pltpu

B, N, C, COUT, E = 2, 512, 128, 128, 18
BI = 16
R = BI * N


def _proj_kernel(x_ref, w_ref, b_ref, out_ref):
    x2 = x_ref[...].reshape(B * N, C)
    out_ref[...] = (jnp.dot(x2, w_ref[...], preferred_element_type=jnp.float32)
                    + b_ref[...]).reshape(B, N, 384)


def _edge_kernel(proj_ref, adj_ref, ef_ref,
                 we1_ref, be1_ref, we2_ref, be2_ref,
                 wa1e_ref, ba1_ref, wa2_ref, ba2_ref,
                 wg1e_ref, bg1_ref, wg2_ref, bg2_ref,
                 wc1s_ref, wc1m_ref, bc1_ref, wc2_ref, bc2_ref,
                 out_ref):
    ib = pl.program_id(1)
    self_feat = proj_ref[0, pl.ds(ib * BI, BI), 0:COUT]
    T = proj_ref[0, :, COUT:2 * COUT]
    ai = proj_ref[0, pl.ds(ib * BI, BI), 2 * COUT:2 * COUT + 32]
    aj = proj_ref[0, :, 2 * COUT + 32:2 * COUT + 64]
    gx = proj_ref[0, :, 2 * COUT + 64:2 * COUT + 128]

    ef = ef_ref[0].reshape(R, E)
    pe1 = jax.nn.relu(jnp.dot(ef, we1_ref[...],
                              preferred_element_type=jnp.float32) + be1_ref[...])
    pe = jax.nn.relu(jnp.dot(pe1, we2_ref[...],
                             preferred_element_type=jnp.float32) + be2_ref[...])

    he = jnp.dot(pe, wa1e_ref[...], preferred_element_type=jnp.float32)
    h3 = jax.nn.relu(he.reshape(BI, N, 32)
                     + ai[:, None, :] + aj[None, :, :] + ba1_ref[...])
    logits = jnp.dot(h3.reshape(R, 32), wa2_ref[...],
                     preferred_element_type=jnp.float32) + ba2_ref[...]
    logits3 = logits.reshape(BI, N, 1)

    mask3 = adj_ref[0].reshape(BI, N, 1) > 0.0
    neg = jnp.where(mask3, logits3, -1e30)
    mx = jnp.max(neg, axis=1, keepdims=True)
    ex = jnp.exp(neg - mx) * mask3.astype(jnp.float32)
    denom = jnp.maximum(jnp.sum(ex, axis=1, keepdims=True), 1e-12)
    att3 = ex / denom

    ge = jnp.dot(pe, wg1e_ref[...], preferred_element_type=jnp.float32)
    gh = jax.nn.relu(ge.reshape(BI, N, 64) + gx[None, :, :] + bg1_ref[...])
    G = jax.nn.sigmoid(jnp.dot(gh.reshape(R, 64), wg2_ref[...],
                               preferred_element_type=jnp.float32) + bg2_ref[...])

    m3 = jnp.sum(att3 * G.reshape(BI, N, COUT) * T[None, :, :], axis=1)
    has_nb = jnp.max(mask3.astype(jnp.float32), axis=1).reshape(BI, 1)
    messages = m3.reshape(BI, COUT) * has_nb

    c1 = jax.nn.relu(jnp.dot(self_feat, wc1s_ref[...],
                             preferred_element_type=jnp.float32)
                     + jnp.dot(messages, wc1m_ref[...],
                               preferred_element_type=jnp.float32)
                     + bc1_ref[...])
    out_ref[0] = (jnp.dot(c1, wc2_ref[...],
                          preferred_element_type=jnp.float32) + bc2_ref[...])


@jax.jit
def kernel(x, adjacency, edge_features, Ws, bs, Wn, bn, We1, be1, We2, be2,
           Wa1, ba1, Wa2, ba2, Wg1, bg1, Wg2, bg2, Wc1, bc1, Wc2, bc2):
    W_all = jnp.concatenate(
        [Ws, Wn, Wa1[:C], Wa1[C:2 * C], Wg1[:C]], axis=1)
    b_all = jnp.concatenate(
        [bs, bn, jnp.zeros((32 + 32 + 64,), jnp.float32)])[None, :]
    proj = pl.pallas_call(
        _proj_kernel,
        out_shape=jax.ShapeDtypeStruct((B, N, 384), jnp.float32),
    )(x, W_all, b_all)

    adj_e = adjacency.reshape(B, N * N, 1)
    row2 = lambda a: a.reshape(1, -1)

    grid = (B, N // BI)
    wspec = pl.BlockSpec(index_map=lambda b, i: (0, 0))
    out = pl.pallas_call(
        _edge_kernel,
        grid=grid,
        in_specs=[
            pl.BlockSpec((1, N, 384), lambda b, i: (b, 0, 0)),
            pl.BlockSpec((1, R, 1), lambda b, i: (b, i, 0)),
            pl.BlockSpec((1, BI, N, E), lambda b, i: (b, i, 0, 0)),
        ] + [wspec] * 17,
        out_specs=pl.BlockSpec((1, BI, COUT), lambda b, i: (b, i, 0)),
        out_shape=jax.ShapeDtypeStruct((B, N, COUT), jnp.float32),
        compiler_params=pltpu.CompilerParams(
            dimension_semantics=("arbitrary", "arbitrary"),
            vmem_limit_bytes=100 * 1024 * 1024,
        ),
    )(proj, adj_e, edge_features,
      We1, row2(be1), We2, row2(be2),
      Wa1[2 * C:], row2(ba1), Wa2, row2(ba2),
      Wg1[C:], row2(bg1), Wg2, row2(bg2),
      Wc1[:COUT], Wc1[COUT:], row2(bc1), Wc2, row2(bc2))
    return out


# speedup vs baseline: 1.7702x; 1.7702x over previous
import jax
import jax.numpy as jnp
from jax.experimental import pallas as pl
from jax.experimental.pallas import tpu as pltpu

B, N, C, COUT, E = 2, 512, 128, 128, 18
BI = 16
R = BI * N


def _proj_kernel(x_ref, w_ref, b_ref, out_ref):
    x2 = x_ref[...].reshape(B * N, C)
    out_ref[...] = (jnp.dot(x2, w_ref[...], preferred_element_type=jnp.float32)
                    + b_ref[...]).reshape(B, N, 384)


def _edge_kernel(proj_ref, adj_ref, ef_ref,
                 we1_ref, be1_ref, we2_ref, be2_ref,
                 wa1e_ref, ba1_ref, wa2_ref, ba2_ref,
                 wg1e_ref, bg1_ref, wg2_ref, bg2_ref,
                 wc1s_ref, wc1m_ref, bc1_ref, wc2_ref, bc2_ref,
                 out_ref):
    ib = pl.program_id(1)
    self_feat = proj_ref[0, pl.ds(ib * BI, BI), 0:COUT]
    T = proj_ref[0, :, COUT:2 * COUT]
    ai = proj_ref[0, pl.ds(ib * BI, BI), 2 * COUT:2 * COUT + 32]
    aj = proj_ref[0, :, 2 * COUT + 32:2 * COUT + 64]
    gx = proj_ref[0, :, 2 * COUT + 64:2 * COUT + 128]

    ef = ef_ref[0].reshape(R, E)
    pe1 = jax.nn.relu(jnp.dot(ef, we1_ref[...],
                              preferred_element_type=jnp.float32) + be1_ref[...])
    pe = jax.nn.relu(jnp.dot(pe1, we2_ref[...],
                             preferred_element_type=jnp.float32) + be2_ref[...])

    he = jnp.dot(pe, wa1e_ref[...],
                 preferred_element_type=jnp.float32) + ba1_ref[...]
    h3 = jax.nn.relu(he.reshape(BI, N, 32) + ai[:, None, :] + aj[None, :, :])
    logits = jnp.dot(h3.reshape(R, 32), wa2_ref[...],
                     preferred_element_type=jnp.float32) + ba2_ref[...]

    logits2 = logits.reshape(BI, N)
    mask2 = adj_ref[0] > 0.0
    neg = jnp.where(mask2, logits2, -1e30)
    mx = jnp.max(neg, axis=1, keepdims=True)
    ex = jnp.exp(neg - mx) * mask2.astype(jnp.float32)
    denom = jnp.maximum(jnp.sum(ex, axis=1, keepdims=True), 1e-12)
    has_nb = jnp.max(mask2.astype(jnp.float32), axis=1, keepdims=True)
    att = (ex / denom).reshape(R, 1)

    ge = jnp.dot(pe, wg1e_ref[...],
                 preferred_element_type=jnp.float32) + bg1_ref[...]
    gh = jax.nn.relu(ge.reshape(BI, N, 64) + gx[None, :, :])
    G = jax.nn.sigmoid(jnp.dot(gh.reshape(R, 64), wg2_ref[...],
                               preferred_element_type=jnp.float32) + bg2_ref[...])

    wg3 = G.reshape(BI, N, COUT) * att.reshape(BI, N, 1)
    m3 = jnp.sum(wg3 * T[None, :, :], axis=1)
    messages = m3.reshape(BI, COUT) * has_nb

    c1 = jax.nn.relu(jnp.dot(self_feat, wc1s_ref[...],
                             preferred_element_type=jnp.float32)
                     + jnp.dot(messages, wc1m_ref[...],
                               preferred_element_type=jnp.float32)
                     + bc1_ref[...])
    out_ref[0] = (jnp.dot(c1, wc2_ref[...],
                          preferred_element_type=jnp.float32) + bc2_ref[...])


@jax.jit
def kernel(x, adjacency, edge_features, Ws, bs, Wn, bn, We1, be1, We2, be2,
           Wa1, ba1, Wa2, ba2, Wg1, bg1, Wg2, bg2, Wc1, bc1, Wc2, bc2):
    W_all = jnp.concatenate(
        [Ws, Wn, Wa1[:C], Wa1[C:2 * C], Wg1[:C]], axis=1)
    b_all = jnp.concatenate(
        [bs, bn, jnp.zeros((32 + 32 + 64,), jnp.float32)])[None, :]
    proj = pl.pallas_call(
        _proj_kernel,
        out_shape=jax.ShapeDtypeStruct((B, N, 384), jnp.float32),
    )(x, W_all, b_all)

    row2 = lambda a: a.reshape(1, -1)

    grid = (B, N // BI)
    wspec = pl.BlockSpec(index_map=lambda b, i: (0, 0))
    out = pl.pallas_call(
        _edge_kernel,
        grid=grid,
        in_specs=[
            pl.BlockSpec((1, N, 384), lambda b, i: (b, 0, 0)),
            pl.BlockSpec((1, BI, N), lambda b, i: (b, i, 0)),
            pl.BlockSpec((1, BI, N, E), lambda b, i: (b, i, 0, 0)),
        ] + [wspec] * 17,
        out_specs=pl.BlockSpec((1, BI, COUT), lambda b, i: (b, i, 0)),
        out_shape=jax.ShapeDtypeStruct((B, N, COUT), jnp.float32),
        compiler_params=pltpu.CompilerParams(
            dimension_semantics=("arbitrary", "arbitrary"),
            vmem_limit_bytes=100 * 1024 * 1024,
        ),
    )(proj, adjacency, edge_features,
      We1, row2(be1), We2, row2(be2),
      Wa1[2 * C:], row2(ba1), Wa2, row2(ba2),
      Wg1[C:], row2(bg1), Wg2, row2(bg2),
      Wc1[:COUT], Wc1[COUT:], row2(bc1), Wc2, row2(bc2))
    return out


# speedup vs baseline: 1.9293x; 1.0899x over previous
import jax
import jax.numpy as jnp
from jax.experimental import pallas as pl
from jax.experimental.pallas import tpu as pltpu

B, N, C, COUT, E = 2, 512, 128, 128, 18
BI = 32
R = BI * N
PJ = 448


def _proj_kernel(x_ref, w_ref, b_ref, out_ref):
    x2 = x_ref[...].reshape(B * N, C)
    out_ref[...] = (jnp.dot(x2, w_ref[...], preferred_element_type=jnp.float32)
                    + b_ref[...]).reshape(B, N, PJ)


def _edge_kernel(proj_ref, adj_ref, ef_ref,
                 we1_ref, be1_ref, we2_ref, be2_ref,
                 whg_ref, wa2p_ref, ba2_ref, wg2p_ref, bg2_ref,
                 wc1s_ref, wc1m_ref, bc1_ref, wc2_ref, bc2_ref,
                 out_ref):
    ib = pl.program_id(1)
    self_feat = proj_ref[0, pl.ds(ib * BI, BI), 0:COUT]
    T = proj_ref[0, :, COUT:2 * COUT]
    aig = proj_ref[0, pl.ds(ib * BI, BI), 256:352]
    ajg = proj_ref[0, :, 352:448]

    ef = ef_ref[0].reshape(R, E)
    pe1 = jax.nn.relu(jnp.dot(ef, we1_ref[...],
                              preferred_element_type=jnp.float32) + be1_ref[...])
    pe = jax.nn.relu(jnp.dot(pe1, we2_ref[...],
                             preferred_element_type=jnp.float32) + be2_ref[...])

    hgm = jnp.dot(pe, whg_ref[...], preferred_element_type=jnp.float32)
    hg3 = jax.nn.relu(hgm.reshape(BI, N, 96) + aig[:, None, :] + ajg[None, :, :])
    hgf = hg3.reshape(R, 96)

    logits = jnp.dot(hgf, wa2p_ref[...],
                     preferred_element_type=jnp.float32) + ba2_ref[...]
    logits2 = logits.reshape(BI, N)
    mask2 = adj_ref[0] > 0.0
    neg = jnp.where(mask2, logits2, -1e30)
    mx = jnp.max(neg, axis=1, keepdims=True)
    ex = jnp.exp(neg - mx) * mask2.astype(jnp.float32)
    denom = jnp.maximum(jnp.sum(ex, axis=1, keepdims=True), 1e-12)
    has_nb = jnp.max(mask2.astype(jnp.float32), axis=1, keepdims=True)
    att = ex / denom

    G = jax.nn.sigmoid(jnp.dot(hgf[:, 0:64], wg2p_ref[...],
                               preferred_element_type=jnp.float32) + bg2_ref[...])

    att_t = jnp.concatenate([att] * BI, axis=1)
    lane_i = jax.lax.broadcasted_iota(jnp.int32, (BI, R), 1) // N
    sub_i = jax.lax.broadcasted_iota(jnp.int32, (BI, R), 0)
    a_blk = jnp.where(lane_i == sub_i, att_t, 0.0)
    gt = (G.reshape(BI, N, COUT) * T[None, :, :]).reshape(R, COUT)
    m3 = jnp.dot(a_blk, gt, preferred_element_type=jnp.float32)
    messages = m3 * has_nb

    c1 = jax.nn.relu(jnp.dot(self_feat, wc1s_ref[...],
                             preferred_element_type=jnp.float32)
                     + jnp.dot(messages, wc1m_ref[...],
                               preferred_element_type=jnp.float32)
                     + bc1_ref[...])
    out_ref[0] = (jnp.dot(c1, wc2_ref[...],
                          preferred_element_type=jnp.float32) + bc2_ref[...])


@jax.jit
def kernel(x, adjacency, edge_features, Ws, bs, Wn, bn, We1, be1, We2, be2,
           Wa1, ba1, Wa2, ba2, Wg1, bg1, Wg2, bg2, Wc1, bc1, Wc2, bc2):
    z = jnp.zeros
    f32 = jnp.float32
    W_all = jnp.concatenate(
        [Ws, Wn, z((C, 64), f32), Wa1[:C], Wg1[:C], Wa1[C:2 * C]], axis=1)
    b_all = jnp.concatenate(
        [bs, bn, z((64,), f32), z((32,), f32), bg1, ba1])[None, :]
    proj = pl.pallas_call(
        _proj_kernel,
        out_shape=jax.ShapeDtypeStruct((B, N, PJ), jnp.float32),
    )(x, W_all, b_all)

    Whg = jnp.concatenate([Wg1[C:], Wa1[2 * C:]], axis=1)
    Wa2p = jnp.concatenate([z((64, 1), f32), Wa2], axis=0)
    Wg2p = Wg2

    row2 = lambda a: a.reshape(1, -1)

    grid = (B, N // BI)
    wspec = pl.BlockSpec(index_map=lambda b, i: (0, 0))
    out = pl.pallas_call(
        _edge_kernel,
        grid=grid,
        in_specs=[
            pl.BlockSpec((1, N, PJ), lambda b, i: (b, 0, 0)),
            pl.BlockSpec((1, BI, N), lambda b, i: (b, i, 0)),
            pl.BlockSpec((1, BI, N, E), lambda b, i: (b, i, 0, 0)),
        ] + [wspec] * 14,
        out_specs=pl.BlockSpec((1, BI, COUT), lambda b, i: (b, i, 0)),
        out_shape=jax.ShapeDtypeStruct((B, N, COUT), jnp.float32),
        compiler_params=pltpu.CompilerParams(
            dimension_semantics=("arbitrary", "arbitrary"),
            vmem_limit_bytes=100 * 1024 * 1024,
        ),
    )(proj, adjacency, edge_features,
      We1, row2(be1), We2, row2(be2),
      Whg, Wa2p, row2(ba2), Wg2p, row2(bg2),
      Wc1[:COUT], Wc1[COUT:], row2(bc1), Wc2, row2(bc2))
    return out


# speedup vs baseline: 2.7473x; 1.4239x over previous
import jax
import jax.numpy as jnp
from jax.experimental import pallas as pl
from jax.experimental.pallas import tpu as pltpu

B, N, C, COUT, E = 2, 512, 128, 128, 18
BI = 32
CH = BI * N
PJ = 448


def _proj_kernel(x_ref, w_ref, b_ref, row_ref, t_ref):
    x2 = x_ref[...].reshape(B * N, C)
    p = (jnp.dot(x2, w_ref[...], preferred_element_type=jnp.float32)
         + b_ref[...]).reshape(B, N, PJ)
    row_ref[...] = p[:, :, 0:224]
    for b in range(B):
        t_ref[b] = jnp.transpose(p[b, :, 224:PJ])


def _edge_kernel(self_ref, pt_ref, adj_ref, ef_ref, sit_ref,
                 we1t_ref, be1_ref, we2t_ref, be2_ref,
                 whgt_ref, wa2pt_ref, ba2_ref, wg2t_ref, bg2_ref,
                 wc1s_ref, wc1m_ref, bc1_ref, wc2_ref, bc2_ref,
                 out_ref):
    ib = pl.program_id(1)
    f32 = jnp.float32

    ef = ef_ref[0]
    pe1 = jax.nn.relu(jnp.dot(we1t_ref[...], ef,
                              preferred_element_type=f32) + be1_ref[...])
    pe = jax.nn.relu(jnp.dot(we2t_ref[...], pe1,
                             preferred_element_type=f32) + be2_ref[...])

    hgm = jnp.dot(whgt_ref[...], pe, preferred_element_type=f32)

    aig = jnp.transpose(self_ref[0, :, 128:224])
    sub_i = jax.lax.broadcasted_iota(jnp.int32, (BI, CH), 0)
    lane_g = jax.lax.broadcasted_iota(jnp.int32, (BI, CH), 1) // N
    s_i = (sub_i == lane_g).astype(f32)
    add_i = jnp.dot(aig, s_i, preferred_element_type=f32)
    ajg = pt_ref[0, 128:224, :]
    add_j = jnp.concatenate([ajg] * BI, axis=1)
    hg = jax.nn.relu(hgm + add_i + add_j)

    logits = jnp.dot(wa2pt_ref[...], hg,
                     preferred_element_type=f32) + ba2_ref[...]
    logits2 = logits.reshape(BI, N)
    mask2 = adj_ref[0] > 0.0
    neg = jnp.where(mask2, logits2, -1e30)
    mx = jnp.max(neg, axis=1, keepdims=True)
    ex = jnp.exp(neg - mx) * mask2.astype(f32)
    denom = jnp.maximum(jnp.sum(ex, axis=1, keepdims=True), 1e-12)
    has_nb = jnp.max(mask2.astype(f32), axis=1, keepdims=True)
    att_row = (ex / denom).reshape(1, CH)

    G = jax.nn.sigmoid(jnp.dot(wg2t_ref[...], hg[0:64, :],
                               preferred_element_type=f32) + bg2_ref[...])

    t_tile = jnp.concatenate([pt_ref[0, 0:128, :]] * BI, axis=1)
    z = G * att_row * t_tile
    m_t = jnp.dot(z, sit_ref[...], preferred_element_type=f32)
    messages = jnp.transpose(m_t) * has_nb

    c1 = jax.nn.relu(jnp.dot(self_ref[0, :, 0:COUT], wc1s_ref[...],
                             preferred_element_type=f32)
                     + jnp.dot(messages, wc1m_ref[...],
                               preferred_element_type=f32)
                     + bc1_ref[...])
    out_ref[0] = (jnp.dot(c1, wc2_ref[...],
                          preferred_element_type=f32) + bc2_ref[...])


@jax.jit
def kernel(x, adjacency, edge_features, Ws, bs, Wn, bn, We1, be1, We2, be2,
           Wa1, ba1, Wa2, ba2, Wg1, bg1, Wg2, bg2, Wc1, bc1, Wc2, bc2):
    z = jnp.zeros
    f32 = jnp.float32
    W_all = jnp.concatenate(
        [Ws, z((C, 64), f32), Wa1[:C], Wn, Wg1[:C], Wa1[C:2 * C]], axis=1)
    b_all = jnp.concatenate(
        [bs, z((64,), f32), z((32,), f32), bn, bg1, ba1])[None, :]
    proj_row, proj_t = pl.pallas_call(
        _proj_kernel,
        out_shape=(jax.ShapeDtypeStruct((B, N, 224), f32),
                   jax.ShapeDtypeStruct((B, 224, N), f32)),
    )(x, W_all, b_all)

    whgt = jnp.concatenate([Wg1[C:], Wa1[2 * C:]], axis=1).T
    wa2pt = jnp.concatenate([z((64, 1), f32), Wa2], axis=0).T
    sit = jnp.repeat(jnp.eye(BI, dtype=f32), N, axis=0)

    ef_r = edge_features.transpose(0, 3, 1, 2).reshape(B, E, N * N)

    col = lambda a: a.reshape(-1, 1)
    row2 = lambda a: a.reshape(1, -1)

    grid = (B, N // BI)
    wspec = pl.BlockSpec(index_map=lambda b, i: (0, 0))
    out = pl.pallas_call(
        _edge_kernel,
        grid=grid,
        in_specs=[
            pl.BlockSpec((1, BI, 224), lambda b, i: (b, i, 0)),
            pl.BlockSpec((1, 224, N), lambda b, i: (b, 0, 0)),
            pl.BlockSpec((1, BI, N), lambda b, i: (b, i, 0)),
            pl.BlockSpec((1, E, CH), lambda b, i: (b, 0, i)),
            wspec,
        ] + [wspec] * 14,
        out_specs=pl.BlockSpec((1, BI, COUT), lambda b, i: (b, i, 0)),
        out_shape=jax.ShapeDtypeStruct((B, N, COUT), f32),
        compiler_params=pltpu.CompilerParams(
            dimension_semantics=("arbitrary", "arbitrary"),
            vmem_limit_bytes=100 * 1024 * 1024,
        ),
    )(proj_row, proj_t, adjacency, ef_r, sit,
      We1.T, col(be1), We2.T, col(be2),
      whgt, wa2pt, col(ba2), Wg2.T, col(bg2),
      Wc1[:COUT], Wc1[COUT:], row2(bc1), Wc2, row2(bc2))
    return out


# speedup vs baseline: 2.8677x; 1.0438x over previous
import jax
import jax.numpy as jnp
from jax.experimental import pallas as pl
from jax.experimental.pallas import tpu as pltpu

B, N, C, COUT, E = 2, 512, 128, 128, 18
BI = 32
CH = BI * N
PJ = 448


def _proj_kernel(x_ref, w_ref, b_ref, row_ref, t_ref):
    x2 = x_ref[...].reshape(B * N, C)
    p = (jnp.dot(x2, w_ref[...], preferred_element_type=jnp.float32)
         + b_ref[...]).reshape(B, N, PJ)
    row_ref[...] = p[:, :, 0:224]
    for b in range(B):
        t_ref[b] = jnp.transpose(p[b, :, 224:PJ])


def _edge_kernel(self_ref, pt_ref, adj_ref, ef_ref, sit_ref,
                 we1t_ref, be1_ref, we2t_ref, be2_ref,
                 whgt_ref, wa2pt_ref, ba2_ref, wg2t_ref, bg2_ref,
                 wc1s_ref, wc1m_ref, bc1_ref, wc2_ref, bc2_ref,
                 out_ref):
    ib = pl.program_id(1)
    f32 = jnp.float32
    bf16 = jnp.bfloat16

    ef = ef_ref[0].astype(bf16)
    pe1 = jax.nn.relu(jnp.dot(we1t_ref[...], ef,
                              preferred_element_type=f32) + be1_ref[...])
    pe = jax.nn.relu(jnp.dot(we2t_ref[...], pe1.astype(bf16),
                             preferred_element_type=f32) + be2_ref[...])

    hgm = jnp.dot(whgt_ref[...], pe.astype(bf16),
                  preferred_element_type=f32)

    aig = jnp.transpose(self_ref[0, :, 128:224])
    sub_i = jax.lax.broadcasted_iota(jnp.int32, (BI, CH), 0)
    lane_g = jax.lax.broadcasted_iota(jnp.int32, (BI, CH), 1) // N
    s_i = (sub_i == lane_g).astype(f32)
    add_i = jnp.dot(aig, s_i, preferred_element_type=f32)
    ajg = pt_ref[0, 128:224, :]
    add_j = jnp.concatenate([ajg] * BI, axis=1)
    hg = jax.nn.relu(hgm + add_i + add_j)

    logits = jnp.dot(wa2pt_ref[...], hg,
                     preferred_element_type=f32) + ba2_ref[...]
    logits2 = logits.reshape(BI, N)
    mask2 = adj_ref[0] > 0.0
    neg = jnp.where(mask2, logits2, -1e30)
    mx = jnp.max(neg, axis=1, keepdims=True)
    ex = jnp.exp(neg - mx) * mask2.astype(f32)
    denom = jnp.maximum(jnp.sum(ex, axis=1, keepdims=True), 1e-12)
    has_nb = jnp.max(mask2.astype(f32), axis=1, keepdims=True)
    att_row = (ex / denom).reshape(1, CH)

    G = jax.nn.sigmoid(jnp.dot(wg2t_ref[...], hg[0:64, :].astype(bf16),
                               preferred_element_type=f32) + bg2_ref[...])

    t_tile = jnp.concatenate([pt_ref[0, 0:128, :]] * BI, axis=1)
    z = G * att_row * t_tile
    m_t = jnp.dot(z.astype(bf16), sit_ref[...],
                  preferred_element_type=f32)
    messages = jnp.transpose(m_t) * has_nb

    c1 = jax.nn.relu(jnp.dot(self_ref[0, :, 0:COUT], wc1s_ref[...],
                             preferred_element_type=f32)
                     + jnp.dot(messages, wc1m_ref[...],
                               preferred_element_type=f32)
                     + bc1_ref[...])
    out_ref[0] = (jnp.dot(c1, wc2_ref[...],
                          preferred_element_type=f32) + bc2_ref[...])


@jax.jit
def kernel(x, adjacency, edge_features, Ws, bs, Wn, bn, We1, be1, We2, be2,
           Wa1, ba1, Wa2, ba2, Wg1, bg1, Wg2, bg2, Wc1, bc1, Wc2, bc2):
    z = jnp.zeros
    f32 = jnp.float32
    W_all = jnp.concatenate(
        [Ws, z((C, 64), f32), Wa1[:C], Wn, Wg1[:C], Wa1[C:2 * C]], axis=1)
    b_all = jnp.concatenate(
        [bs, z((64,), f32), z((32,), f32), bn, bg1, ba1])[None, :]
    proj_row, proj_t = pl.pallas_call(
        _proj_kernel,
        out_shape=(jax.ShapeDtypeStruct((B, N, 224), f32),
                   jax.ShapeDtypeStruct((B, 224, N), f32)),
    )(x, W_all, b_all)

    whgt = jnp.concatenate([Wg1[C:], Wa1[2 * C:]], axis=1).T
    wa2pt = jnp.concatenate([z((64, 1), f32), Wa2], axis=0).T
    sit = jnp.repeat(jnp.eye(BI, dtype=f32), N, axis=0)

    ef_r = edge_features.transpose(0, 3, 1, 2).reshape(B, E, N * N)

    col = lambda a: a.reshape(-1, 1)
    row2 = lambda a: a.reshape(1, -1)

    grid = (B, N // BI)
    wspec = pl.BlockSpec(index_map=lambda b, i: (0, 0))
    out = pl.pallas_call(
        _edge_kernel,
        grid=grid,
        in_specs=[
            pl.BlockSpec((1, BI, 224), lambda b, i: (b, i, 0)),
            pl.BlockSpec((1, 224, N), lambda b, i: (b, 0, 0)),
            pl.BlockSpec((1, BI, N), lambda b, i: (b, i, 0)),
            pl.BlockSpec((1, E, CH), lambda b, i: (b, 0, i)),
            wspec,
        ] + [wspec] * 14,
        out_specs=pl.BlockSpec((1, BI, COUT), lambda b, i: (b, i, 0)),
        out_shape=jax.ShapeDtypeStruct((B, N, COUT), f32),
        compiler_params=pltpu.CompilerParams(
            dimension_semantics=("arbitrary", "arbitrary"),
            vmem_limit_bytes=100 * 1024 * 1024,
        ),
    )(proj_row, proj_t, adjacency, ef_r, sit.astype(jnp.bfloat16),
      We1.T.astype(jnp.bfloat16), col(be1), We2.T.astype(jnp.bfloat16),
      col(be2), whgt.astype(jnp.bfloat16), wa2pt, col(ba2),
      Wg2.T.astype(jnp.bfloat16), col(bg2),
      Wc1[:COUT], Wc1[COUT:], row2(bc1), Wc2, row2(bc2))
    return out


# speedup vs baseline: 3.2989x; 1.1504x over previous
import jax
import jax.numpy as jnp
from jax.experimental import pallas as pl
from jax.experimental.pallas import tpu as pltpu

B, N, C, COUT, E = 2, 512, 128, 128, 18
BI = 32
CH = BI * N
PJ = 448


def _proj_kernel(x_ref, w_ref, b_ref, row_ref, t_ref):
    x2 = x_ref[...].reshape(B * N, C)
    p = (jnp.dot(x2, w_ref[...], preferred_element_type=jnp.float32)
         + b_ref[...]).reshape(B, N, PJ)
    row_ref[...] = p[:, :, 0:224]
    for b in range(B):
        t_ref[b] = jnp.transpose(p[b, :, 224:PJ])


def _edge_kernel(self_ref, pt_ref, adj_ref, ef_ref, sit_ref,
                 we1t_ref, be1_ref, we2t_ref, be2_ref,
                 whgt_ref, wa2pt_ref, ba2_ref, wg2t_ref, bg2_ref,
                 wc1s_ref, wc1m_ref, bc1_ref, wc2_ref, bc2_ref,
                 out_ref, buf, sem):
    ib = pl.program_id(1)
    f32 = jnp.float32
    bf16 = jnp.bfloat16

    nib = N // BI
    p = pl.program_id(0) * nib + ib
    nprog = B * nib
    slot = jax.lax.rem(p, 2)
    nslot = 1 - slot

    def block_copies(pp, s):
        b2 = pp // nib
        ib2 = jax.lax.rem(pp, nib)
        return [pltpu.make_async_copy(
            ef_ref.at[b2, :, ib2 * BI + k, :],
            buf.at[s, :, pl.ds(k * N, N)],
            sem.at[s]) for k in range(BI)]

    @pl.when(p == 0)
    def _():
        for c in block_copies(0, 0):
            c.start()

    @pl.when(p + 1 < nprog)
    def _():
        for c in block_copies(p + 1, nslot):
            c.start()

    for c in block_copies(p, slot):
        c.wait()

    ef = buf[slot].astype(bf16)
    pe1 = jax.nn.relu(jnp.dot(we1t_ref[...], ef,
                              preferred_element_type=f32) + be1_ref[...])
    pe = jax.nn.relu(jnp.dot(we2t_ref[...], pe1.astype(bf16),
                             preferred_element_type=f32) + be2_ref[...])

    hgm = jnp.dot(whgt_ref[...], pe.astype(bf16),
                  preferred_element_type=f32)

    aig = jnp.transpose(self_ref[0, :, 128:224])
    sub_i = jax.lax.broadcasted_iota(jnp.int32, (BI, CH), 0)
    lane_g = jax.lax.broadcasted_iota(jnp.int32, (BI, CH), 1) // N
    s_i = (sub_i == lane_g).astype(f32)
    add_i = jnp.dot(aig, s_i, preferred_element_type=f32)
    ajg = pt_ref[0, 128:224, :]
    add_j = jnp.concatenate([ajg] * BI, axis=1)
    hg = jax.nn.relu(hgm + add_i + add_j)

    logits = jnp.dot(wa2pt_ref[...], hg,
                     preferred_element_type=f32) + ba2_ref[...]
    logits2 = logits.reshape(BI, N)
    mask2 = adj_ref[0] > 0.0
    neg = jnp.where(mask2, logits2, -1e30)
    mx = jnp.max(neg, axis=1, keepdims=True)
    ex = jnp.exp(neg - mx) * mask2.astype(f32)
    denom = jnp.maximum(jnp.sum(ex, axis=1, keepdims=True), 1e-12)
    has_nb = jnp.max(mask2.astype(f32), axis=1, keepdims=True)
    att_row = (ex / denom).reshape(1, CH)

    G = jax.nn.sigmoid(jnp.dot(wg2t_ref[...], hg[0:64, :].astype(bf16),
                               preferred_element_type=f32) + bg2_ref[...])

    t_tile = jnp.concatenate([pt_ref[0, 0:128, :]] * BI, axis=1)
    z = G * att_row * t_tile
    m_t = jnp.dot(z.astype(bf16), sit_ref[...],
                  preferred_element_type=f32)
    messages = jnp.transpose(m_t) * has_nb

    c1 = jax.nn.relu(jnp.dot(self_ref[0, :, 0:COUT], wc1s_ref[...],
                             preferred_element_type=f32)
                     + jnp.dot(messages, wc1m_ref[...],
                               preferred_element_type=f32)
                     + bc1_ref[...])
    out_ref[0] = (jnp.dot(c1, wc2_ref[...],
                          preferred_element_type=f32) + bc2_ref[...])


@jax.jit
def kernel(x, adjacency, edge_features, Ws, bs, Wn, bn, We1, be1, We2, be2,
           Wa1, ba1, Wa2, ba2, Wg1, bg1, Wg2, bg2, Wc1, bc1, Wc2, bc2):
    z = jnp.zeros
    f32 = jnp.float32
    W_all = jnp.concatenate(
        [Ws, z((C, 64), f32), Wa1[:C], Wn, Wg1[:C], Wa1[C:2 * C]], axis=1)
    b_all = jnp.concatenate(
        [bs, z((64,), f32), z((32,), f32), bn, bg1, ba1])[None, :]
    proj_row, proj_t = pl.pallas_call(
        _proj_kernel,
        out_shape=(jax.ShapeDtypeStruct((B, N, 224), f32),
                   jax.ShapeDtypeStruct((B, 224, N), f32)),
    )(x, W_all, b_all)

    whgt = jnp.concatenate([Wg1[C:], Wa1[2 * C:]], axis=1).T
    wa2pt = jnp.concatenate([z((64, 1), f32), Wa2], axis=0).T
    sit = jnp.repeat(jnp.eye(BI, dtype=f32), N, axis=0)

    ef_t = edge_features.transpose(0, 3, 1, 2)

    col = lambda a: a.reshape(-1, 1)
    row2 = lambda a: a.reshape(1, -1)

    grid = (B, N // BI)
    wspec = pl.BlockSpec(index_map=lambda b, i: (0, 0))
    out = pl.pallas_call(
        _edge_kernel,
        grid=grid,
        in_specs=[
            pl.BlockSpec((1, BI, 224), lambda b, i: (b, i, 0)),
            pl.BlockSpec((1, 224, N), lambda b, i: (b, 0, 0)),
            pl.BlockSpec((1, BI, N), lambda b, i: (b, i, 0)),
            pl.BlockSpec(memory_space=pltpu.MemorySpace.HBM),
            wspec,
        ] + [wspec] * 14,
        out_specs=pl.BlockSpec((1, BI, COUT), lambda b, i: (b, i, 0)),
        out_shape=jax.ShapeDtypeStruct((B, N, COUT), f32),
        scratch_shapes=[pltpu.VMEM((2, E, CH), f32),
                        pltpu.SemaphoreType.DMA((2,))],
        compiler_params=pltpu.CompilerParams(
            dimension_semantics=("arbitrary", "arbitrary"),
            vmem_limit_bytes=100 * 1024 * 1024,
        ),
    )(proj_row, proj_t, adjacency, ef_t, sit.astype(jnp.bfloat16),
      We1.T.astype(jnp.bfloat16), col(be1), We2.T.astype(jnp.bfloat16),
      col(be2), whgt.astype(jnp.bfloat16), wa2pt, col(ba2),
      Wg2.T.astype(jnp.bfloat16), col(bg2),
      Wc1[:COUT], Wc1[COUT:], row2(bc1), Wc2, row2(bc2))
    return out


# speedup vs baseline: 3.4668x; 1.0509x over previous
import jax
import jax.numpy as jnp
from jax.experimental import pallas as pl
from jax.experimental.pallas import tpu as pltpu

B, N, C, COUT, E = 2, 512, 128, 128, 18
BI = 32
CH = BI * N
PJ = 448


def _prep_kernel(x_ref, ws_ref, bs_ref, wn_ref, bn_ref, we1_ref, be1_ref,
                 we2_ref, be2_ref, wa1_ref, ba1_ref, wa2_ref, wg1_ref,
                 bg1_ref, wg2_ref, bg2_ref, wc1_ref,
                 row_ref, t_ref, sit_ref, we1t_ref, we2t_ref, whgt_ref,
                 wg2t_ref, wa2pt_ref, be1c_ref, be2c_ref, bg2c_ref,
                 wc1s_ref, wc1m_ref):
    f32 = jnp.float32
    bf16 = jnp.bfloat16
    zc = jnp.zeros((C, 96), f32)
    w_all = jnp.concatenate(
        [ws_ref[...], zc[:, 0:64], wa1_ref[0:C], wn_ref[...],
         wg1_ref[0:C], wa1_ref[C:2 * C]], axis=1)
    b_all = jnp.concatenate(
        [bs_ref[...], jnp.zeros((1, 96), f32), bn_ref[...],
         bg1_ref[...], ba1_ref[...]], axis=1)
    x2 = x_ref[...].reshape(B * N, C)
    p = (jnp.dot(x2, w_all, preferred_element_type=f32)
         + b_all).reshape(B, N, PJ)
    row_ref[...] = p[:, :, 0:224]
    for b in range(B):
        t_ref[b] = jnp.transpose(p[b, :, 224:PJ])

    sub = jax.lax.broadcasted_iota(jnp.int32, (CH, BI), 0) // N
    lane = jax.lax.broadcasted_iota(jnp.int32, (CH, BI), 1)
    sit_ref[...] = (sub == lane).astype(bf16)

    we1t_ref[...] = jnp.transpose(we1_ref[...]).astype(bf16)
    we2t_ref[...] = jnp.transpose(we2_ref[...]).astype(bf16)
    whg = jnp.concatenate([wg1_ref[C:C + 32], wa1_ref[2 * C:]], axis=1)
    whgt_ref[...] = jnp.transpose(whg).astype(bf16)
    wg2t_ref[...] = jnp.transpose(wg2_ref[...]).astype(bf16)
    wa2p = jnp.concatenate([jnp.zeros((64, 1), f32), wa2_ref[...]], axis=0)
    wa2pt_ref[...] = jnp.transpose(wa2p)
    be1c_ref[...] = jnp.transpose(be1_ref[...])
    be2c_ref[...] = jnp.transpose(be2_ref[...])
    bg2c_ref[...] = jnp.transpose(bg2_ref[...])
    wc1s_ref[...] = wc1_ref[0:COUT]
    wc1m_ref[...] = wc1_ref[COUT:]


def _edge_kernel(self_ref, pt_ref, adj_ref, ef_ref, sit_ref,
                 we1t_ref, be1_ref, we2t_ref, be2_ref,
                 whgt_ref, wa2pt_ref, ba2_ref, wg2t_ref, bg2_ref,
                 wc1s_ref, wc1m_ref, bc1_ref, wc2_ref, bc2_ref,
                 out_ref, buf, sem):
    ib = pl.program_id(1)
    f32 = jnp.float32
    bf16 = jnp.bfloat16

    nib = N // BI
    p = pl.program_id(0) * nib + ib
    nprog = B * nib
    slot = jax.lax.rem(p, 2)
    nslot = 1 - slot

    def block_copies(pp, s):
        b2 = pp // nib
        ib2 = jax.lax.rem(pp, nib)
        return [pltpu.make_async_copy(
            ef_ref.at[b2, :, ib2 * BI + k, :],
            buf.at[s, :, pl.ds(k * N, N)],
            sem.at[s]) for k in range(BI)]

    @pl.when(p == 0)
    def _():
        for c in block_copies(0, 0):
            c.start()

    @pl.when(p + 1 < nprog)
    def _():
        for c in block_copies(p + 1, nslot):
            c.start()

    for c in block_copies(p, slot):
        c.wait()

    ef = buf[slot].astype(bf16)
    pe1 = jax.nn.relu(jnp.dot(we1t_ref[...], ef,
                              preferred_element_type=f32) + be1_ref[...])
    pe = jax.nn.relu(jnp.dot(we2t_ref[...], pe1.astype(bf16),
                             preferred_element_type=f32) + be2_ref[...])

    hgm = jnp.dot(whgt_ref[...], pe.astype(bf16),
                  preferred_element_type=f32)

    aig = jnp.transpose(self_ref[0, :, 128:224])
    sub_i = jax.lax.broadcasted_iota(jnp.int32, (BI, CH), 0)
    lane_g = jax.lax.broadcasted_iota(jnp.int32, (BI, CH), 1) // N
    s_i = (sub_i == lane_g).astype(f32)
    add_i = jnp.dot(aig, s_i, preferred_element_type=f32)
    ajg = pt_ref[0, 128:224, :]
    add_j = jnp.concatenate([ajg] * BI, axis=1)
    hg = jax.nn.relu(hgm + add_i + add_j)

    logits = jnp.dot(wa2pt_ref[...], hg,
                     preferred_element_type=f32) + ba2_ref[...]
    logits2 = logits.reshape(BI, N)
    mask2 = adj_ref[0] > 0.0
    neg = jnp.where(mask2, logits2, -1e30)
    mx = jnp.max(neg, axis=1, keepdims=True)
    ex = jnp.exp(neg - mx) * mask2.astype(f32)
    denom = jnp.maximum(jnp.sum(ex, axis=1, keepdims=True), 1e-12)
    has_nb = jnp.max(mask2.astype(f32), axis=1, keepdims=True)
    att_row = (ex / denom).reshape(1, CH)

    G = jax.nn.sigmoid(jnp.dot(wg2t_ref[...], hg[0:64, :].astype(bf16),
                               preferred_element_type=f32) + bg2_ref[...])

    t_tile = jnp.concatenate([pt_ref[0, 0:128, :]] * BI, axis=1)
    z = G * att_row * t_tile
    m_t = jnp.dot(z.astype(bf16), sit_ref[...],
                  preferred_element_type=f32)
    messages = jnp.transpose(m_t) * has_nb

    c1 = jax.nn.relu(jnp.dot(self_ref[0, :, 0:COUT], wc1s_ref[...],
                             preferred_element_type=f32)
                     + jnp.dot(messages, wc1m_ref[...],
                               preferred_element_type=f32)
                     + bc1_ref[...])
    out_ref[0] = (jnp.dot(c1, wc2_ref[...],
                          preferred_element_type=f32) + bc2_ref[...])


@jax.jit
def kernel(x, adjacency, edge_features, Ws, bs, Wn, bn, We1, be1, We2, be2,
           Wa1, ba1, Wa2, ba2, Wg1, bg1, Wg2, bg2, Wc1, bc1, Wc2, bc2):
    f32 = jnp.float32
    bf16 = jnp.bfloat16
    row2 = lambda a: a.reshape(1, -1)
    sds = jax.ShapeDtypeStruct
    (proj_row, proj_t, sit, we1t, we2t, whgt, wg2t, wa2pt,
     be1c, be2c, bg2c, wc1s, wc1m) = pl.pallas_call(
        _prep_kernel,
        out_shape=(sds((B, N, 224), f32), sds((B, 224, N), f32),
                   sds((CH, BI), bf16), sds((32, E), bf16),
                   sds((32, 32), bf16), sds((96, 32), bf16),
                   sds((COUT, 64), bf16), sds((1, 96), f32),
                   sds((32, 1), f32), sds((32, 1), f32),
                   sds((COUT, 1), f32), sds((COUT, COUT), f32),
                   sds((COUT, COUT), f32)),
    )(x, Ws, row2(bs), Wn, row2(bn), We1, row2(be1), We2, row2(be2),
      Wa1, row2(ba1), Wa2, Wg1, row2(bg1), Wg2, row2(bg2), Wc1)

    ef_t = edge_features.transpose(0, 3, 1, 2)

    grid = (B, N // BI)
    wspec = pl.BlockSpec(index_map=lambda b, i: (0, 0))
    out = pl.pallas_call(
        _edge_kernel,
        grid=grid,
        in_specs=[
            pl.BlockSpec((1, BI, 224), lambda b, i: (b, i, 0)),
            pl.BlockSpec((1, 224, N), lambda b, i: (b, 0, 0)),
            pl.BlockSpec((1, BI, N), lambda b, i: (b, i, 0)),
            pl.BlockSpec(memory_space=pltpu.MemorySpace.HBM),
            wspec,
        ] + [wspec] * 14,
        out_specs=pl.BlockSpec((1, BI, COUT), lambda b, i: (b, i, 0)),
        out_shape=jax.ShapeDtypeStruct((B, N, COUT), f32),
        scratch_shapes=[pltpu.VMEM((2, E, CH), f32),
                        pltpu.SemaphoreType.DMA((2,))],
        compiler_params=pltpu.CompilerParams(
            dimension_semantics=("arbitrary", "arbitrary"),
            vmem_limit_bytes=100 * 1024 * 1024,
        ),
    )(proj_row, proj_t, adjacency, ef_t, sit,
      we1t, be1c, we2t, be2c,
      whgt, wa2pt, ba2.reshape(1, 1), wg2t, bg2c,
      wc1s, wc1m, row2(bc1), Wc2, row2(bc2))
    return out


# speedup vs baseline: 3.5382x; 1.0206x over previous
import jax
import jax.numpy as jnp
from jax.experimental import pallas as pl
from jax.experimental.pallas import tpu as pltpu

B, N, C, COUT, E = 2, 512, 128, 128, 18
BI = 64
CH = BI * N
PJ = 448


def _prep_kernel(x_ref, ws_ref, bs_ref, wn_ref, bn_ref, we1_ref, be1_ref,
                 we2_ref, be2_ref, wa1_ref, ba1_ref, wa2_ref, wg1_ref,
                 bg1_ref, wg2_ref, bg2_ref, wc1_ref,
                 row_ref, t_ref, ajg_ref, sit_ref, we1t_ref, we2t_ref,
                 whgt_ref, wg2t_ref, wa2pt_ref, be1c_ref, be2c_ref, bg2c_ref,
                 wc1s_ref, wc1m_ref):
    f32 = jnp.float32
    bf16 = jnp.bfloat16
    zc = jnp.zeros((C, 96), f32)
    w_all = jnp.concatenate(
        [ws_ref[...], zc[:, 0:64], wa1_ref[0:C], wn_ref[...],
         wg1_ref[0:C], wa1_ref[C:2 * C]], axis=1)
    b_all = jnp.concatenate(
        [bs_ref[...], jnp.zeros((1, 96), f32), bn_ref[...],
         bg1_ref[...], ba1_ref[...]], axis=1)
    x2 = x_ref[...].reshape(B * N, C)
    p = (jnp.dot(x2, w_all, preferred_element_type=f32)
         + b_all).reshape(B, N, PJ)
    row_ref[...] = p[:, :, 0:224]
    for b in range(B):
        tb = jnp.transpose(p[b, :, 224:PJ])
        t_ref[b] = tb[0:COUT].astype(bf16)
        ajg_ref[b] = tb[COUT:224]

    sub = jax.lax.broadcasted_iota(jnp.int32, (CH, BI), 0) // N
    lane = jax.lax.broadcasted_iota(jnp.int32, (CH, BI), 1)
    sit_ref[...] = (sub == lane).astype(bf16)

    we1t_ref[...] = jnp.transpose(we1_ref[...]).astype(bf16)
    we2t_ref[...] = jnp.transpose(we2_ref[...]).astype(bf16)
    whg = jnp.concatenate([wg1_ref[C:C + 32], wa1_ref[2 * C:]], axis=1)
    whgt_ref[...] = jnp.transpose(whg).astype(bf16)
    wg2t_ref[...] = jnp.transpose(wg2_ref[...]).astype(bf16)
    wa2p = jnp.concatenate([jnp.zeros((64, 1), f32), wa2_ref[...]], axis=0)
    wa2pt_ref[...] = jnp.transpose(wa2p).astype(bf16)
    be1c_ref[...] = jnp.transpose(be1_ref[...])
    be2c_ref[...] = jnp.transpose(be2_ref[...])
    bg2c_ref[...] = jnp.transpose(bg2_ref[...])
    wc1s_ref[...] = wc1_ref[0:COUT]
    wc1m_ref[...] = wc1_ref[COUT:]


def _edge_kernel(self_ref, pt_ref, ajg_ref, adj_ref, ef_ref, sit_ref,
                 we1t_ref, be1_ref, we2t_ref, be2_ref,
                 whgt_ref, wa2pt_ref, ba2_ref, wg2t_ref, bg2_ref,
                 wc1s_ref, wc1m_ref, bc1_ref, wc2_ref, bc2_ref,
                 out_ref, buf, sem):
    ib = pl.program_id(1)
    f32 = jnp.float32
    bf16 = jnp.bfloat16

    nib = N // BI
    p = pl.program_id(0) * nib + ib
    nprog = B * nib
    slot = jax.lax.rem(p, 2)
    nslot = 1 - slot

    def block_copies(pp, s):
        b2 = pp // nib
        ib2 = jax.lax.rem(pp, nib)
        return [pltpu.make_async_copy(
            ef_ref.at[b2, :, ib2 * BI + k, :],
            buf.at[s, :, pl.ds(k * N, N)],
            sem.at[s]) for k in range(BI)]

    @pl.when(p == 0)
    def _():
        for c in block_copies(0, 0):
            c.start()

    @pl.when(p + 1 < nprog)
    def _():
        for c in block_copies(p + 1, nslot):
            c.start()

    for c in block_copies(p, slot):
        c.wait()

    ef = buf[slot].astype(bf16)
    pe1 = jax.nn.relu(jnp.dot(we1t_ref[...], ef,
                              preferred_element_type=f32)
                      + be1_ref[...]).astype(bf16)
    pe = jax.nn.relu(jnp.dot(we2t_ref[...], pe1,
                             preferred_element_type=f32)
                     + be2_ref[...]).astype(bf16)

    hgm = jnp.dot(whgt_ref[...], pe,
                  preferred_element_type=f32)

    aig = jnp.transpose(self_ref[0, :, 128:224]).astype(bf16)
    sub_i = jax.lax.broadcasted_iota(jnp.int32, (BI, CH), 0)
    lane_g = jax.lax.broadcasted_iota(jnp.int32, (BI, CH), 1) // N
    s_i = (sub_i == lane_g).astype(bf16)
    add_i = jnp.dot(aig, s_i, preferred_element_type=f32)
    ajg = ajg_ref[0]
    add_j = jnp.concatenate([ajg] * BI, axis=1)
    hg = jax.nn.relu(hgm + add_i + add_j).astype(bf16)

    logits = jnp.dot(wa2pt_ref[...], hg,
                     preferred_element_type=f32) + ba2_ref[...]
    logits2 = logits.reshape(BI, N)
    mask2 = adj_ref[0] > 0.0
    neg = jnp.where(mask2, logits2, -1e30)
    mx = jnp.max(neg, axis=1, keepdims=True)
    ex = jnp.exp(neg - mx) * mask2.astype(f32)
    denom = jnp.maximum(jnp.sum(ex, axis=1, keepdims=True), 1e-12)
    has_nb = jnp.max(mask2.astype(f32), axis=1, keepdims=True)
    att_row = (ex / denom).reshape(1, CH).astype(bf16)

    G = jax.nn.sigmoid(jnp.dot(wg2t_ref[...], hg[0:64, :],
                               preferred_element_type=f32)
                       + bg2_ref[...]).astype(bf16)

    t_tile = jnp.concatenate([pt_ref[0, 0:128, :]] * BI, axis=1)
    z = G * att_row * t_tile
    m_t = jnp.dot(z, sit_ref[...],
                  preferred_element_type=f32)
    messages = jnp.transpose(m_t) * has_nb

    c1 = jax.nn.relu(jnp.dot(self_ref[0, :, 0:COUT], wc1s_ref[...],
                             preferred_element_type=f32)
                     + jnp.dot(messages, wc1m_ref[...],
                               preferred_element_type=f32)
                     + bc1_ref[...])
    out_ref[0] = (jnp.dot(c1, wc2_ref[...],
                          preferred_element_type=f32) + bc2_ref[...])


@jax.jit
def kernel(x, adjacency, edge_features, Ws, bs, Wn, bn, We1, be1, We2, be2,
           Wa1, ba1, Wa2, ba2, Wg1, bg1, Wg2, bg2, Wc1, bc1, Wc2, bc2):
    f32 = jnp.float32
    bf16 = jnp.bfloat16
    row2 = lambda a: a.reshape(1, -1)
    sds = jax.ShapeDtypeStruct
    (proj_row, proj_t, proj_ajg, sit, we1t, we2t, whgt, wg2t, wa2pt,
     be1c, be2c, bg2c, wc1s, wc1m) = pl.pallas_call(
        _prep_kernel,
        out_shape=(sds((B, N, 224), f32), sds((B, COUT, N), bf16),
                   sds((B, 96, N), f32),
                   sds((CH, BI), bf16), sds((32, E), bf16),
                   sds((32, 32), bf16), sds((96, 32), bf16),
                   sds((COUT, 64), bf16), sds((1, 96), bf16),
                   sds((32, 1), f32), sds((32, 1), f32),
                   sds((COUT, 1), f32), sds((COUT, COUT), f32),
                   sds((COUT, COUT), f32)),
    )(x, Ws, row2(bs), Wn, row2(bn), We1, row2(be1), We2, row2(be2),
      Wa1, row2(ba1), Wa2, Wg1, row2(bg1), Wg2, row2(bg2), Wc1)

    ef_t = edge_features.transpose(0, 3, 1, 2)

    grid = (B, N // BI)
    wspec = pl.BlockSpec(index_map=lambda b, i: (0, 0))
    out = pl.pallas_call(
        _edge_kernel,
        grid=grid,
        in_specs=[
            pl.BlockSpec((1, BI, 224), lambda b, i: (b, i, 0)),
            pl.BlockSpec((1, COUT, N), lambda b, i: (b, 0, 0)),
            pl.BlockSpec((1, 96, N), lambda b, i: (b, 0, 0)),
            pl.BlockSpec((1, BI, N), lambda b, i: (b, i, 0)),
            pl.BlockSpec(memory_space=pltpu.MemorySpace.HBM),
            wspec,
        ] + [wspec] * 14,
        out_specs=pl.BlockSpec((1, BI, COUT), lambda b, i: (b, i, 0)),
        out_shape=jax.ShapeDtypeStruct((B, N, COUT), f32),
        scratch_shapes=[pltpu.VMEM((2, E, CH), f32),
                        pltpu.SemaphoreType.DMA((2,))],
        compiler_params=pltpu.CompilerParams(
            dimension_semantics=("arbitrary", "arbitrary"),
            vmem_limit_bytes=100 * 1024 * 1024,
        ),
    )(proj_row, proj_t, proj_ajg, adjacency, ef_t, sit,
      we1t, be1c, we2t, be2c,
      whgt, wa2pt, ba2.reshape(1, 1), wg2t, bg2c,
      wc1s, wc1m, row2(bc1), Wc2, row2(bc2))
    return out
